# baseline (speedup 1.0000x reference)
import functools

import jax
import jax.numpy as jnp
from jax import lax
from jax.experimental import pallas as pl
from jax.experimental.pallas import tpu as pltpu
from jax.experimental.pallas import tpu_sc as plsc

N = 10000
R = 10240
E = 320000
CHUNK = 128
DP = 128
NCORE = 2
NSUB = 16
NW = NCORE * NSUB
NBUF = 2
IDXD = 4
CPW = 80
EPAD = CPW * NW * CHUNK
ROWS_PW = R // NSUB
BR = 1024


def _fill(buf_slot, val):
    v16 = jnp.full((16,), val, jnp.float32)

    def row(r, carry):
        for cv in range(DP // 16):
            buf_slot[r, pl.ds(cv * 16, 16)] = v16
        return carry

    lax.fori_loop(0, CHUNK, row, 0)


def _clear_acc(bufs, acc, s):
    _fill(bufs.at[0], 0.0)
    for k in range(ROWS_PW // CHUNK):
        pltpu.sync_copy(bufs.at[0], acc.at[pl.ds(s * ROWS_PW + k * CHUNK, CHUNK)])


def _spmm_body(p_hbm, idx_hbm, out_hbm, idx_v, bufs, acc, i0, i1, i2, i3, g0, g1):
    isems = (i0, i1, i2, i3)
    gsems = (g0, g1)
    c = lax.axis_index("c")
    s = lax.axis_index("s")
    _clear_acc(bufs, acc, s)
    plsc.subcore_barrier()

    for q in range(IDXD):
        pltpu.async_copy(idx_hbm.at[c, s, q], idx_v.at[q], isems[q])
    for b in range(NBUF):
        pltpu.make_async_copy(idx_hbm.at[c, s, b], idx_v.at[b], isems[b]).wait()
        pltpu.async_copy(p_hbm.at[idx_v.at[b, 0]], bufs.at[b], gsems[b])

    def step(j, b, ib, do_idx, do_gather):
        pltpu.make_async_copy(p_hbm.at[idx_v.at[ib, 0]], bufs.at[b],
                              gsems[b]).wait()
        pltpu.sync_copy(bufs.at[b], acc.at[idx_v.at[ib, 1]], add=True)
        if do_idx:
            pltpu.async_copy(idx_hbm.at[c, s, j + IDXD], idx_v.at[ib], isems[ib])
        if do_gather:
            ib2 = (ib + NBUF) % IDXD
            pltpu.make_async_copy(idx_hbm.at[c, s, j + NBUF], idx_v.at[ib2],
                                  isems[ib2]).wait()
            pltpu.async_copy(p_hbm.at[idx_v.at[ib2, 0]], bufs.at[b], gsems[b])

    def super_round(k, carry):
        base = k * IDXD
        for q in range(IDXD):
            step(base + q, q % NBUF, q, True, True)
        return carry

    lax.fori_loop(0, CPW // IDXD - 1, super_round, 0)
    for q in range(IDXD):
        j = CPW - IDXD + q
        step(j, q % NBUF, q, False, j + NBUF < CPW)

    plsc.subcore_barrier()
    pltpu.sync_copy(acc.at[pl.ds(s * ROWS_PW, ROWS_PW)],
                    out_hbm.at[c, pl.ds(s * ROWS_PW, ROWS_PW)])


@functools.lru_cache(maxsize=None)
def _make_spmm():
    mesh = plsc.VectorSubcoreMesh(core_axis_name="c", subcore_axis_name="s")
    return functools.partial(
        pl.kernel,
        mesh=mesh,
        out_type=jax.ShapeDtypeStruct((NCORE, R, DP), jnp.float32),
        scratch_types=[
            pltpu.VMEM((IDXD, 2, CHUNK), jnp.int32),
            pltpu.VMEM((NBUF, CHUNK, DP), jnp.float32),
            pltpu.VMEM_SHARED((R, DP), jnp.float32),
            pltpu.SemaphoreType.DMA,
            pltpu.SemaphoreType.DMA,
            pltpu.SemaphoreType.DMA,
            pltpu.SemaphoreType.DMA,
            pltpu.SemaphoreType.DMA,
            pltpu.SemaphoreType.DMA,
        ],
    )(_spmm_body)


def _dinv_of(deg0_ref, deg1_ref):
    return lax.rsqrt(deg0_ref[:, 0:1] + deg1_ref[:, 0:1] + 1.0)


def _first_body(x_ref, w_ref, deg0_ref, deg1_ref, out_ref):
    dinv = _dinv_of(deg0_ref, deg1_ref)
    out_ref[...] = jnp.dot(x_ref[...], w_ref[...],
                           preferred_element_type=jnp.float32) * dinv


def _mid_body(s0_ref, s1_ref, p_ref, deg0_ref, deg1_ref, b_ref, w_ref, out_ref,
              *, act):
    dinv = _dinv_of(deg0_ref, deg1_ref)
    h = (s0_ref[...] + s1_ref[...] + p_ref[...]) * dinv + b_ref[...]
    if act == "relu":
        h = jnp.maximum(h, 0.0)
    out_ref[...] = jnp.dot(h, w_ref[...],
                           preferred_element_type=jnp.float32) * dinv


def _last_body(s0_ref, s1_ref, p_ref, deg0_ref, deg1_ref, b_ref, out_ref, *, act):
    dinv = _dinv_of(deg0_ref, deg1_ref)
    h = (s0_ref[...] + s1_ref[...] + p_ref[...]) * dinv + b_ref[...]
    if act == "relu":
        h = jnp.maximum(h, 0.0)
    else:
        h = jax.nn.sigmoid(h)
    out_ref[...] = h


def _row_spec(d):
    return pl.BlockSpec((BR, d), lambda i: (i, 0))


def _full_spec(r, c):
    return pl.BlockSpec((r, c), lambda i: (0, 0))


def _tc_first(xp, w, deg0, deg1):
    dout = w.shape[1]
    return pl.pallas_call(
        _first_body,
        grid=(R // BR,),
        in_specs=[_row_spec(xp.shape[1]), _full_spec(*w.shape),
                  _row_spec(DP), _row_spec(DP)],
        out_specs=_row_spec(dout),
        out_shape=jax.ShapeDtypeStruct((R, dout), jnp.float32),
    )(xp, w, deg0, deg1)


def _tc_mid(s, p, deg0, deg1, b, w, act):
    dprev = p.shape[1]
    dout = w.shape[1]
    return pl.pallas_call(
        functools.partial(_mid_body, act=act),
        grid=(R // BR,),
        in_specs=[_row_spec(dprev), _row_spec(dprev), _row_spec(dprev),
                  _row_spec(DP), _row_spec(DP),
                  _full_spec(1, dprev), _full_spec(*w.shape)],
        out_specs=_row_spec(dout),
        out_shape=jax.ShapeDtypeStruct((R, dout), jnp.float32),
    )(s[0], s[1], p, deg0, deg1, b, w)


def _tc_last(s, p, deg0, deg1, b, act):
    dprev = p.shape[1]
    return pl.pallas_call(
        functools.partial(_last_body, act=act),
        grid=(R // BR,),
        in_specs=[_row_spec(dprev), _row_spec(dprev), _row_spec(dprev),
                  _row_spec(DP), _row_spec(DP), _full_spec(1, dprev)],
        out_specs=_row_spec(dprev),
        out_shape=jax.ShapeDtypeStruct((R, dprev), jnp.float32),
    )(s[0], s[1], p, deg0, deg1, b)


def _pad_w(w):
    return jnp.pad(w, ((0, DP - w.shape[0]), (0, DP - w.shape[1])))


def _pad_b(b):
    return jnp.pad(b, (0, DP - b.shape[0])).reshape(1, DP)


def _chunk_idx(src, dst):
    def one(v):
        pad = jnp.full((EPAD - E,), N, v.dtype)
        return jnp.concatenate([v, pad]).reshape(NCORE, NSUB, CPW, 1, CHUNK)

    return jnp.concatenate([one(src), one(dst)], axis=3)


def kernel(x, edge_index, W1e, b1e, W2e, b2e, Wmue, bmue, Wlse, blse, W4e, b4e,
           W1n, b1n, Wmun, bmun, Wlsn, blsn, W5n, b5n):
    del Wlse, blse, Wlsn, blsn
    xp = jnp.pad(x, ((0, R - N), (0, 0)))
    idx = _chunk_idx(edge_index[0].astype(jnp.int32),
                     edge_index[1].astype(jnp.int32))

    def spmm(p):
        return _make_spmm()(p, idx)

    degS = spmm(jnp.ones((R, DP), jnp.float32))
    deg0, deg1 = degS[0], degS[1]

    p = _tc_first(xp, _pad_w(W1e), deg0, deg1)
    s = spmm(p)
    p = _tc_mid(s, p, deg0, deg1, _pad_b(b1e), _pad_w(W2e), "relu")
    s = spmm(p)
    p = _tc_mid(s, p, deg0, deg1, _pad_b(b2e), _pad_w(Wmue), "relu")
    s = spmm(p)
    p = _tc_mid(s, p, deg0, deg1, _pad_b(bmue), _pad_w(W4e), "id")
    s = spmm(p)
    edges = _tc_last(s, p, deg0, deg1, _pad_b(b4e), "sigmoid")[:N, :16]

    p = _tc_first(xp, W1n, deg0, deg1)
    s = spmm(p)
    p = _tc_mid(s, p, deg0, deg1, _pad_b(b1n), Wmun, "relu")
    s = spmm(p)
    p = _tc_mid(s, p, deg0, deg1, _pad_b(bmun), W5n, "id")
    s = spmm(p)
    nodes = _tc_last(s, p, deg0, deg1, _pad_b(b5n), "relu")[:N]

    return (edges, nodes)
